# baseline (speedup 1.0000x reference)
import functools

import jax
import jax.numpy as jnp
from jax import lax
from jax.experimental import pallas as pl
from jax.experimental.pallas import tpu as pltpu
from jax.experimental.pallas import tpu_sc as plsc

N = 50000
E = 1600000
B = 32
NUM_LABELS = 20

NC = 2
NS = 16
NW = NC * NS
L = 16

K = 128
CHT = 400
NG = CHT // 8
NGA = 60
NGB = 2 * NG - NGA
E_PAD = NW * K * CHT
N_PAD = 50176
ZROWS = N_PAD // NS
CPIECE = N_PAD // 8

_f32 = jnp.float32
_i32 = jnp.int32


def _sc_body(xT, srcs, dsts, alphas, biases, z32,
             out_acc,
             acc_sh, src8, dst8, al8, bi8, grows, tidx,
             isem0, isem1,
             gsem0, gsem1, gsem2, gsem3,
             ssem0, ssem1, ssem2, ssem3):
    isem = [isem0, isem1]
    gsem = [gsem0, gsem1, gsem2, gsem3]
    ssem = [ssem0, ssem1, ssem2, ssem3]

    c_ax = lax.axis_index("c")
    s_ax = lax.axis_index("s")
    is0 = c_ax == 0
    gbase = jnp.where(is0, s_ax * NGA, NS * NGA + s_ax * NGB)
    ng_self = jnp.where(is0, NGA, NGB)
    ngpair = jnp.where(is0, NGA // 2, NGB // 2)

    zbase = s_ax * ZROWS
    pltpu.sync_copy(z32, acc_sh.at[pl.ds(zbase, ZROWS)])
    for r in range(K // L):
        tidx[pl.ds(r * L, L)] = jnp.full((L,), N, dtype=_i32)
    plsc.subcore_barrier()

    def grp_issue(p, g):
        gg = jnp.minimum(gbase + g, gbase + ng_self - 1)
        pltpu.async_copy(srcs.at[gg], src8.at[p], isem[p])
        pltpu.async_copy(dsts.at[gg], dst8.at[p], isem[p])
        pltpu.async_copy(alphas.at[gg], al8.at[p], isem[p])
        pltpu.async_copy(biases.at[gg], bi8.at[p], isem[p])

    def grp_wait(p):
        pltpu.make_async_copy(srcs.at[gbase], src8.at[p], isem[p]).wait()
        pltpu.make_async_copy(dsts.at[gbase], dst8.at[p], isem[p]).wait()
        pltpu.make_async_copy(alphas.at[gbase], al8.at[p], isem[p]).wait()
        pltpu.make_async_copy(biases.at[gbase], bi8.at[p], isem[p]).wait()

    def gather_issue(p, row, buf):
        pltpu.async_copy(xT.at[src8.at[p, row]], grows.at[buf], gsem[buf])

    def gather_wait(p, row, buf):
        pltpu.make_async_copy(xT.at[src8.at[p, row]], grows.at[buf],
                              gsem[buf]).wait()

    def scatter_issue(p, row, buf):
        pltpu.async_copy(grows.at[buf], acc_sh.at[dst8.at[p, row]],
                         ssem[buf], add=True)

    def scatter_wait(p, row, buf):
        pltpu.make_async_copy(grows.at[buf], acc_sh.at[dst8.at[p, row]],
                              ssem[buf]).wait()

    def compute(p, row, buf):
        def g16(g2, carry):
            a16 = al8[p, row, pl.ds(g2 * L, L)]
            b16 = bi8[p, row, pl.ds(g2 * L, L)]
            for jj in range(L):
                e = g2 * L + jj
                sel = jnp.full((L,), jj, dtype=_i32)
                av = a16.at[sel].get(mode="promise_in_bounds")
                bv = b16.at[sel].get(mode="promise_in_bounds")
                r0 = grows[buf, e, pl.ds(0, L)]
                r1 = grows[buf, e, pl.ds(L, L)]
                grows[buf, e, pl.ds(0, L)] = r0 * av + bv
                grows[buf, e, pl.ds(L, L)] = r1 * av + bv
            return carry
        lax.fori_loop(0, K // L, g16, 0)

    grp_issue(0, 0)
    for b in (2, 3):
        pltpu.async_copy(grows.at[b], acc_sh.at[tidx], ssem[b], add=True)
    grp_wait(0)
    gather_issue(0, 0, 0)
    gather_issue(0, 1, 1)

    def outer(gp, carry):
        for p in range(2):
            g = gp * 2 + p
            for j in range(8):
                b = j % 4
                cc = g * 8 + j
                gather_wait(p, j, b)
                scatter_wait(p, (j + 2) % 8 if j < 6 else j - 6,
                             (j + 2) % 4)
                if j == 4:
                    grp_issue(1 - p, g + 1)
                if j == 6:
                    grp_wait(1 - p)
                if j < 6:
                    gather_issue(p, j + 2, (j + 2) % 4)
                else:
                    gather_issue(1 - p, j - 6, (j + 2) % 4)
                compute(p, j, b)
                scatter_issue(p, j, b)
        return carry

    lax.fori_loop(0, ngpair, outer, 0)

    gather_wait(1, 0, 0)
    gather_wait(1, 1, 1)
    scatter_wait(1, 6, 2)
    scatter_wait(1, 7, 3)

    plsc.subcore_barrier()
    pltpu.sync_copy(acc_sh.at[pl.ds(zbase, ZROWS)],
                    out_acc.at[c_ax, pl.ds(zbase, ZROWS)])


_sc_call = functools.partial(
    pl.kernel,
    out_type=jax.ShapeDtypeStruct((NC, N_PAD, B), _f32),
    mesh=plsc.VectorSubcoreMesh(core_axis_name="c", subcore_axis_name="s",
                                num_cores=NC, num_subcores=NS),
    scratch_types=(
        [pltpu.VMEM_SHARED((N_PAD, B), _f32),
         pltpu.VMEM((2, 8, K), _i32),
         pltpu.VMEM((2, 8, K), _i32),
         pltpu.VMEM((2, 8, K), _f32),
         pltpu.VMEM((2, 8, K), _f32),
         pltpu.VMEM((4, K, B), _f32),
         pltpu.VMEM((K,), _i32)]
        + [pltpu.SemaphoreType.DMA] * 10
    ),
    compiler_params=pltpu.CompilerParams(use_tc_tiling_on_sc=False,
                                         needs_layout_passes=False),
)(_sc_body)


DBLK = 1600
NBLK = CHT * K // DBLK


def _cnt_body(dsts_flat, out_cnt, cnt_local, dbuf, csem0, csem1):
    csem = [csem0, csem1]
    c_ax = lax.axis_index("c")
    s_ax = lax.axis_index("s")
    wid = c_ax * NS + s_ax
    ebase = wid * CHT * K

    def zcnt(i, carry):
        cnt_local[pl.ds(i * L, L)] = jnp.zeros((L,), _f32)
        return carry
    lax.fori_loop(0, N_PAD // L, zcnt, 0)

    def blk_issue(p, i):
        eb = ebase + jnp.minimum(i, NBLK - 1) * DBLK
        pltpu.async_copy(dsts_flat.at[pl.ds(eb, DBLK)], dbuf.at[p],
                         csem[p])

    def blk_wait(p):
        pltpu.make_async_copy(dsts_flat.at[pl.ds(ebase, DBLK)], dbuf.at[p],
                              csem[p]).wait()

    blk_issue(0, 0)

    def blk2(i2, carry):
        for p in range(2):
            i = i2 * 2 + p
            blk_issue(1 - p, i + 1)
            blk_wait(p)

            def g16(g2, carry2):
                d16 = dbuf[p, pl.ds(g2 * L, L)]
                dcnt, dlast = plsc.scan_count(d16)
                plsc.addupdate_scatter(cnt_local, [d16],
                                       dcnt.astype(_f32), mask=dlast)
                return carry2
            lax.fori_loop(0, DBLK // L, g16, 0)
        return carry

    lax.fori_loop(0, NBLK // 2, blk2, 0)
    blk_wait(0)

    def cout(i, carry):
        pltpu.sync_copy(cnt_local.at[pl.ds(i * CPIECE, CPIECE)],
                        out_cnt.at[c_ax, s_ax, pl.ds(i * CPIECE, CPIECE)])
        return carry
    lax.fori_loop(0, N_PAD // CPIECE, cout, 0)


_cnt_call = functools.partial(
    pl.kernel,
    out_type=jax.ShapeDtypeStruct((NC, NS, N_PAD), _f32),
    mesh=plsc.VectorSubcoreMesh(core_axis_name="c", subcore_axis_name="s",
                                num_cores=NC, num_subcores=NS),
    scratch_types=[
        pltpu.VMEM((N_PAD,), _f32),
        pltpu.VMEM((2, DBLK), _i32),
        pltpu.SemaphoreType.DMA,
        pltpu.SemaphoreType.DMA,
    ],
    compiler_params=pltpu.CompilerParams(use_tc_tiling_on_sc=False,
                                         needs_layout_passes=False),
)(_cnt_body)


def _tr_body(x_ref, out_ref):
    r = lax.broadcasted_iota(_i32, (B, B), 0)
    cidx = lax.broadcasted_iota(_i32, (B, B), 1)
    eye = (r == cidx).astype(_f32)
    out_ref[...] = jax.lax.dot_general(
        x_ref[...], eye, (((0,), (0,)), ((), ())),
        preferred_element_type=_f32)


def _transpose_x(xp):
    return pl.pallas_call(
        _tr_body,
        out_shape=jax.ShapeDtypeStruct((N_PAD, B), _f32),
    )(xp)


TBLK = 1792
NB = N_PAD // TBLK


def _fin_body(acc_ref, cnt_ref, w_ref, g_ref, b_ref, bn_ref, pred_ref, pacc):
    i = pl.program_id(0)
    sums = acc_ref[0] + acc_ref[1]
    counts = jnp.sum(cnt_ref[0] + cnt_ref[1], axis=0)[:, None]
    mean = sums / jnp.maximum(counts, 1.0)
    th = jnp.tanh(mean)
    mu = jnp.mean(th, axis=1, keepdims=True)
    var = jnp.mean((th - mu) * (th - mu), axis=1, keepdims=True)
    bn = (th - mu) / jnp.sqrt(var + 1e-5) * g_ref[...] + b_ref[...]

    r = lax.broadcasted_iota(_i32, (B, B), 0)
    cidx = lax.broadcasted_iota(_i32, (B, B), 1)
    eye = (r == cidx).astype(_f32)
    bn_ref[...] = jax.lax.dot_general(
        eye, bn, (((1,), (1,)), ((), ())), preferred_element_type=_f32)

    @pl.when(i == 0)
    def _():
        pacc[...] = jnp.zeros((B, B), _f32)

    pacc[...] += jax.lax.dot_general(
        w_ref[...], mean, (((1,), (0,)), ((), ())),
        preferred_element_type=_f32)

    @pl.when(i == NB - 1)
    def _():
        pred_ref[...] = pacc[...]


def _final(acc, cnt, Wp, g2, b2):
    return pl.pallas_call(
        _fin_body,
        grid=(NB,),
        in_specs=[
            pl.BlockSpec((NC, TBLK, B), lambda i: (0, i, 0)),
            pl.BlockSpec((NC, NS, TBLK), lambda i: (0, 0, i)),
            pl.BlockSpec((B, TBLK), lambda i: (0, i)),
            pl.BlockSpec((TBLK, 1), lambda i: (i, 0)),
            pl.BlockSpec((TBLK, 1), lambda i: (i, 0)),
        ],
        out_specs=[
            pl.BlockSpec((B, TBLK), lambda i: (0, i)),
            pl.BlockSpec((B, B), lambda i: (0, 0)),
        ],
        out_shape=[
            jax.ShapeDtypeStruct((B, N_PAD), _f32),
            jax.ShapeDtypeStruct((B, B), _f32),
        ],
        scratch_shapes=[pltpu.VMEM((B, B), _f32)],
    )(acc, cnt, Wp, g2, b2)


def kernel(x, edge_index, alpha, bias, W, b_lin, gamma, beta):
    src = edge_index[0]
    dst = edge_index[1]
    pad = E_PAD - E
    srcs = jnp.concatenate([src, jnp.zeros((pad,), _i32)]).reshape(-1, 8, K)
    pad_dst = N + (jnp.arange(pad, dtype=_i32) % (N_PAD - N))
    dsts = jnp.concatenate([dst, pad_dst]).reshape(-1, 8, K)
    alphas = jnp.concatenate([alpha, jnp.zeros((pad,), _f32)]
                             ).reshape(-1, 8, K)
    biases = jnp.concatenate([bias, jnp.zeros((pad,), _f32)]
                             ).reshape(-1, 8, K)

    xp = jnp.pad(x, ((0, 0), (0, N_PAD - N)))
    xT = _transpose_x(xp)

    z32 = jnp.zeros((ZROWS, B), _f32)

    acc = _sc_call(xT, srcs, dsts, alphas, biases, z32)
    cnt = _cnt_call(dsts.reshape(-1))

    Wp = jnp.pad(W, ((0, B - NUM_LABELS), (0, N_PAD - N)))
    g2 = jnp.pad(gamma, (0, N_PAD - N))[:, None]
    b2 = jnp.pad(beta, (0, N_PAD - N))[:, None]

    bn_full, pred32 = _final(acc, cnt, Wp, g2, b2)

    pred = pred32.T[:, :NUM_LABELS] + b_lin[None, :]
    bn = bn_full[:, :N]
    return (pred, bn)


# speedup vs baseline: 1.0187x; 1.0187x over previous
import functools

import jax
import jax.numpy as jnp
from jax import lax
from jax.experimental import pallas as pl
from jax.experimental.pallas import tpu as pltpu
from jax.experimental.pallas import tpu_sc as plsc

N = 50000
E = 1600000
B = 32
NUM_LABELS = 20

NC = 2
NS = 16
NW = NC * NS
L = 16

K = 128
CHT = 400
NG = CHT // 8
NGA = 80
NGB = 2 * NG - NGA
E_PAD = NW * K * CHT
N_PAD = 50176
ZROWS = N_PAD // NS
CPIECE = N_PAD // 8

_f32 = jnp.float32
_i32 = jnp.int32


def _sc_body(xT, srcs, dsts, alphas, biases, z32,
             out_acc,
             acc_sh, src8, dst8, al8, bi8, grows, tidx,
             isem0, isem1,
             gsem0, gsem1, gsem2, gsem3,
             ssem0, ssem1, ssem2, ssem3):
    isem = [isem0, isem1]
    gsem = [gsem0, gsem1, gsem2, gsem3]
    ssem = [ssem0, ssem1, ssem2, ssem3]

    c_ax = lax.axis_index("c")
    s_ax = lax.axis_index("s")
    is0 = c_ax == 0
    gbase = jnp.where(is0, s_ax * NGA, NS * NGA + s_ax * NGB)
    ng_self = jnp.where(is0, NGA, NGB)
    ngpair = jnp.where(is0, NGA // 2, NGB // 2)

    zbase = s_ax * ZROWS
    pltpu.sync_copy(z32, acc_sh.at[pl.ds(zbase, ZROWS)])
    for r in range(K // L):
        tidx[pl.ds(r * L, L)] = jnp.full((L,), N, dtype=_i32)
    plsc.subcore_barrier()

    def grp_issue(p, g):
        gg = jnp.minimum(gbase + g, gbase + ng_self - 1)
        pltpu.async_copy(srcs.at[gg], src8.at[p], isem[p])
        pltpu.async_copy(dsts.at[gg], dst8.at[p], isem[p])
        pltpu.async_copy(alphas.at[gg], al8.at[p], isem[p])
        pltpu.async_copy(biases.at[gg], bi8.at[p], isem[p])

    def grp_wait(p):
        pltpu.make_async_copy(srcs.at[gbase], src8.at[p], isem[p]).wait()
        pltpu.make_async_copy(dsts.at[gbase], dst8.at[p], isem[p]).wait()
        pltpu.make_async_copy(alphas.at[gbase], al8.at[p], isem[p]).wait()
        pltpu.make_async_copy(biases.at[gbase], bi8.at[p], isem[p]).wait()

    def gather_issue(p, row, buf):
        pltpu.async_copy(xT.at[src8.at[p, row]], grows.at[buf], gsem[buf])

    def gather_wait(p, row, buf):
        pltpu.make_async_copy(xT.at[src8.at[p, row]], grows.at[buf],
                              gsem[buf]).wait()

    def scatter_issue(p, row, buf):
        pltpu.async_copy(grows.at[buf], acc_sh.at[dst8.at[p, row]],
                         ssem[buf], add=True)

    def scatter_wait(p, row, buf):
        pltpu.make_async_copy(grows.at[buf], acc_sh.at[dst8.at[p, row]],
                              ssem[buf]).wait()

    def compute(p, row, buf):
        def g16(g2, carry):
            a16 = al8[p, row, pl.ds(g2 * L, L)]
            b16 = bi8[p, row, pl.ds(g2 * L, L)]
            for jj in range(L):
                e = g2 * L + jj
                sel = jnp.full((L,), jj, dtype=_i32)
                av = a16.at[sel].get(mode="promise_in_bounds")
                bv = b16.at[sel].get(mode="promise_in_bounds")
                r0 = grows[buf, e, pl.ds(0, L)]
                r1 = grows[buf, e, pl.ds(L, L)]
                grows[buf, e, pl.ds(0, L)] = r0 * av + bv
                grows[buf, e, pl.ds(L, L)] = r1 * av + bv
            return carry
        lax.fori_loop(0, K // L, g16, 0)

    grp_issue(0, 0)
    for b in (2, 3):
        pltpu.async_copy(grows.at[b], acc_sh.at[tidx], ssem[b], add=True)
    grp_wait(0)
    gather_issue(0, 0, 0)
    gather_issue(0, 1, 1)

    def outer(gp, carry):
        for p in range(2):
            g = gp * 2 + p
            for j in range(8):
                b = j % 4
                cc = g * 8 + j
                gather_wait(p, j, b)
                scatter_wait(p, (j + 2) % 8 if j < 6 else j - 6,
                             (j + 2) % 4)
                if j == 4:
                    grp_issue(1 - p, g + 1)
                if j == 6:
                    grp_wait(1 - p)
                if j < 6:
                    gather_issue(p, j + 2, (j + 2) % 4)
                else:
                    gather_issue(1 - p, j - 6, (j + 2) % 4)
                compute(p, j, b)
                scatter_issue(p, j, b)
        return carry

    lax.fori_loop(0, ngpair, outer, 0)

    gather_wait(1, 0, 0)
    gather_wait(1, 1, 1)
    scatter_wait(1, 6, 2)
    scatter_wait(1, 7, 3)

    plsc.subcore_barrier()
    pltpu.sync_copy(acc_sh.at[pl.ds(zbase, ZROWS)],
                    out_acc.at[c_ax, pl.ds(zbase, ZROWS)])


_sc_call = functools.partial(
    pl.kernel,
    out_type=jax.ShapeDtypeStruct((NC, N_PAD, B), _f32),
    mesh=plsc.VectorSubcoreMesh(core_axis_name="c", subcore_axis_name="s",
                                num_cores=NC, num_subcores=NS),
    scratch_types=(
        [pltpu.VMEM_SHARED((N_PAD, B), _f32),
         pltpu.VMEM((2, 8, K), _i32),
         pltpu.VMEM((2, 8, K), _i32),
         pltpu.VMEM((2, 8, K), _f32),
         pltpu.VMEM((2, 8, K), _f32),
         pltpu.VMEM((4, K, B), _f32),
         pltpu.VMEM((K,), _i32)]
        + [pltpu.SemaphoreType.DMA] * 10
    ),
    compiler_params=pltpu.CompilerParams(use_tc_tiling_on_sc=False,
                                         needs_layout_passes=False),
)(_sc_body)


DBLK = 1600
NBLK = CHT * K // DBLK


def _cnt_body(dsts_flat, out_cnt, cnt_local, dbuf, csem0, csem1):
    csem = [csem0, csem1]
    c_ax = lax.axis_index("c")
    s_ax = lax.axis_index("s")
    wid = c_ax * NS + s_ax
    ebase = wid * CHT * K

    def zcnt(i, carry):
        cnt_local[pl.ds(i * L, L)] = jnp.zeros((L,), _f32)
        return carry
    lax.fori_loop(0, N_PAD // L, zcnt, 0)

    def blk_issue(p, i):
        eb = ebase + jnp.minimum(i, NBLK - 1) * DBLK
        pltpu.async_copy(dsts_flat.at[pl.ds(eb, DBLK)], dbuf.at[p],
                         csem[p])

    def blk_wait(p):
        pltpu.make_async_copy(dsts_flat.at[pl.ds(ebase, DBLK)], dbuf.at[p],
                              csem[p]).wait()

    blk_issue(0, 0)

    def blk2(i2, carry):
        for p in range(2):
            i = i2 * 2 + p
            blk_issue(1 - p, i + 1)
            blk_wait(p)

            def g16(g2, carry2):
                d16 = dbuf[p, pl.ds(g2 * L, L)]
                dcnt, dlast = plsc.scan_count(d16)
                plsc.addupdate_scatter(cnt_local, [d16],
                                       dcnt.astype(_f32), mask=dlast)
                return carry2
            lax.fori_loop(0, DBLK // L, g16, 0)
        return carry

    lax.fori_loop(0, NBLK // 2, blk2, 0)
    blk_wait(0)

    def cout(i, carry):
        pltpu.sync_copy(cnt_local.at[pl.ds(i * CPIECE, CPIECE)],
                        out_cnt.at[c_ax, s_ax, pl.ds(i * CPIECE, CPIECE)])
        return carry
    lax.fori_loop(0, N_PAD // CPIECE, cout, 0)


_cnt_call = functools.partial(
    pl.kernel,
    out_type=jax.ShapeDtypeStruct((NC, NS, N_PAD), _f32),
    mesh=plsc.VectorSubcoreMesh(core_axis_name="c", subcore_axis_name="s",
                                num_cores=NC, num_subcores=NS),
    scratch_types=[
        pltpu.VMEM((N_PAD,), _f32),
        pltpu.VMEM((2, DBLK), _i32),
        pltpu.SemaphoreType.DMA,
        pltpu.SemaphoreType.DMA,
    ],
    compiler_params=pltpu.CompilerParams(use_tc_tiling_on_sc=False,
                                         needs_layout_passes=False),
)(_cnt_body)


def _tr_body(x_ref, out_ref):
    r = lax.broadcasted_iota(_i32, (B, B), 0)
    cidx = lax.broadcasted_iota(_i32, (B, B), 1)
    eye = (r == cidx).astype(_f32)
    out_ref[...] = jax.lax.dot_general(
        x_ref[...], eye, (((0,), (0,)), ((), ())),
        preferred_element_type=_f32)


def _transpose_x(xp):
    return pl.pallas_call(
        _tr_body,
        out_shape=jax.ShapeDtypeStruct((N_PAD, B), _f32),
    )(xp)


TBLK = 1792
NB = N_PAD // TBLK


def _fin_body(acc_ref, cnt_ref, w_ref, g_ref, b_ref, bn_ref, pred_ref, pacc):
    i = pl.program_id(0)
    sums = acc_ref[0] + acc_ref[1]
    counts = jnp.sum(cnt_ref[0] + cnt_ref[1], axis=0)[:, None]
    mean = sums / jnp.maximum(counts, 1.0)
    th = jnp.tanh(mean)
    mu = jnp.mean(th, axis=1, keepdims=True)
    var = jnp.mean((th - mu) * (th - mu), axis=1, keepdims=True)
    bn = (th - mu) / jnp.sqrt(var + 1e-5) * g_ref[...] + b_ref[...]

    r = lax.broadcasted_iota(_i32, (B, B), 0)
    cidx = lax.broadcasted_iota(_i32, (B, B), 1)
    eye = (r == cidx).astype(_f32)
    bn_ref[...] = jax.lax.dot_general(
        eye, bn, (((1,), (1,)), ((), ())), preferred_element_type=_f32)

    @pl.when(i == 0)
    def _():
        pacc[...] = jnp.zeros((B, B), _f32)

    pacc[...] += jax.lax.dot_general(
        w_ref[...], mean, (((1,), (0,)), ((), ())),
        preferred_element_type=_f32)

    @pl.when(i == NB - 1)
    def _():
        pred_ref[...] = pacc[...]


def _final(acc, cnt, Wp, g2, b2):
    return pl.pallas_call(
        _fin_body,
        grid=(NB,),
        in_specs=[
            pl.BlockSpec((NC, TBLK, B), lambda i: (0, i, 0)),
            pl.BlockSpec((NC, NS, TBLK), lambda i: (0, 0, i)),
            pl.BlockSpec((B, TBLK), lambda i: (0, i)),
            pl.BlockSpec((TBLK, 1), lambda i: (i, 0)),
            pl.BlockSpec((TBLK, 1), lambda i: (i, 0)),
        ],
        out_specs=[
            pl.BlockSpec((B, TBLK), lambda i: (0, i)),
            pl.BlockSpec((B, B), lambda i: (0, 0)),
        ],
        out_shape=[
            jax.ShapeDtypeStruct((B, N_PAD), _f32),
            jax.ShapeDtypeStruct((B, B), _f32),
        ],
        scratch_shapes=[pltpu.VMEM((B, B), _f32)],
    )(acc, cnt, Wp, g2, b2)


def kernel(x, edge_index, alpha, bias, W, b_lin, gamma, beta):
    src = edge_index[0]
    dst = edge_index[1]
    pad = E_PAD - E
    srcs = jnp.concatenate([src, jnp.zeros((pad,), _i32)]).reshape(-1, 8, K)
    pad_dst = N + (jnp.arange(pad, dtype=_i32) % (N_PAD - N))
    dsts = jnp.concatenate([dst, pad_dst]).reshape(-1, 8, K)
    alphas = jnp.concatenate([alpha, jnp.zeros((pad,), _f32)]
                             ).reshape(-1, 8, K)
    biases = jnp.concatenate([bias, jnp.zeros((pad,), _f32)]
                             ).reshape(-1, 8, K)

    xp = jnp.pad(x, ((0, 0), (0, N_PAD - N)))
    xT = _transpose_x(xp)

    z32 = jnp.zeros((ZROWS, B), _f32)

    acc = _sc_call(xT, srcs, dsts, alphas, biases, z32)
    cnt = _cnt_call(dsts.reshape(-1))

    Wp = jnp.pad(W, ((0, B - NUM_LABELS), (0, N_PAD - N)))
    g2 = jnp.pad(gamma, (0, N_PAD - N))[:, None]
    b2 = jnp.pad(beta, (0, N_PAD - N))[:, None]

    bn_full, pred32 = _final(acc, cnt, Wp, g2, b2)

    pred = pred32.T[:, :NUM_LABELS] + b_lin[None, :]
    bn = bn_full[:, :N]
    return (pred, bn)
